# Initial kernel scaffold; baseline (speedup 1.0000x reference)
import jax
import jax.numpy as jnp
from jax import lax
from jax.experimental import pallas as pl
from jax.experimental.pallas import tpu as pltpu
from jax.experimental.pallas import tpu_sc as plsc

CONF = 0.3
MAX_WH = 4096.0
MAX_DET = 300
N = 20000
NCLS = 15
B = 4
W = 40
ROWS_PER_SUB = 1248
ROWS_MAX = 1280
GROUPS = ROWS_MAX // 16
DET_PAD = 304
NCHUNK = DET_PAD // 16
NEG_INF = float("-inf")


def _iota():
    return lax.iota(jnp.int32, 16)


def _splat_i32(v):
    return jnp.zeros((16,), jnp.int32) + v


def _splat_f32(v):
    return jnp.zeros((16,), jnp.float32) + v


def _cfun(pc_ref, bm_ref, q, t_scalar):
    lo = jnp.zeros((16,), jnp.int32)
    hi = _splat_i32(N)
    nm1 = _splat_i32(N - 1)
    for _ in range(15):
        mid = jnp.minimum(lax.shift_right_logical(lo + hi, 1), nm1)
        pv = plsc.load_gather(pc_ref, [mid])
        cond = pv > q
        hi = jnp.where(cond, mid, hi)
        lo = jnp.where(cond, lo, mid + 1)
    in_t = q < t_scalar
    r = jnp.where(in_t, lo, 0)
    pcm1 = plsc.load_gather(pc_ref, [jnp.maximum(r - 1, 0)])
    pcx = jnp.where(r > 0, pcm1, 0)
    rem = q - pcx
    bmv = plsc.load_gather(bm_ref, [r])
    col = jnp.zeros((16,), jnp.int32)
    seen = jnp.zeros((16,), jnp.int32)
    found = jnp.zeros((16,), jnp.bool_)
    for t in range(NCLS):
        bit = lax.shift_right_logical(bmv, t) & 1
        hit = (bit == 1) & (seen == rem) & jnp.logical_not(found)
        col = jnp.where(hit, t, col)
        found = jnp.logical_or(found, hit)
        seen = seen + bit
    col = jnp.where(in_t, col, 0)
    return r, col


def _sc_body(pred_ref, out_ref, buf, idx1, lbm, lpc, lrank, pc_all, bm_all,
             rank_all, idx3, rowbuf, outbuf, c2buf, tot_v, cnt_all,
             spm_pc, spm_bm, spm_rank, spm_cnt, sem):
    c = lax.axis_index("c")
    s = lax.axis_index("s")
    iv = _iota()
    rowbase = ROWS_PER_SUB * s
    nrows = jnp.where(s == 15, ROWS_MAX, ROWS_PER_SUB)

    for bb in range(2):
        b = 2 * c + bb

        @pl.loop(0, GROUPS)
        def _build_idx(g):
            rid = iv + 16 * g
            idx1[pl.ds(16 * g, 16)] = 5 * (b * N + rowbase + rid)

        for t in range(ROWS_MAX // 128):
            pltpu.async_copy(
                pred_ref.at[idx1.at[pl.ds(128 * t, 128)]],
                buf.at[pl.ds(128 * t, 128), :], sem)
        for t in range(ROWS_MAX // 128):
            pltpu.make_async_copy(
                pred_ref.at[idx1.at[pl.ds(128 * t, 128)]],
                buf.at[pl.ds(128 * t, 128), :], sem).wait()

        def p1_group(g, carry):
            vcar, ccar = carry
            rid = iv + 16 * g
            rmask = rid < nrows
            confv = plsc.load_gather(buf, [rid, _splat_i32(4)])
            mc = (confv > CONF) & rmask
            bmv = jnp.zeros((16,), jnp.int32)
            cntv = jnp.zeros((16,), jnp.int32)
            for j in range(NCLS):
                cv = plsc.load_gather(buf, [rid, _splat_i32(5 + j)])
                vj = mc & (cv * confv > CONF)
                bmv = bmv | jnp.where(vj, 1 << j, 0)
                cntv = cntv + jnp.where(vj, 1, 0)
            lbm[pl.ds(16 * g, 16)] = bmv
            pcv = plsc.cumsum(cntv) + vcar
            lpc[pl.ds(16 * g, 16)] = pcv
            mci = jnp.where(mc, 1, 0)
            rkv = plsc.cumsum(mci) + ccar
            lrank[pl.ds(16 * g, 16)] = rkv
            return vcar + jnp.sum(cntv), ccar + jnp.sum(mci)

        vc_tot, cc_tot = lax.fori_loop(
            0, GROUPS, p1_group, (jnp.int32(0), jnp.int32(0)))

        totv = jnp.where(iv == 0, vc_tot, jnp.where(iv == 1, cc_tot, 0))
        tot_v[...] = totv
        pltpu.sync_copy(tot_v, spm_cnt.at[s])
        plsc.subcore_barrier()

        pltpu.sync_copy(spm_cnt, cnt_all)
        vc_col = plsc.load_gather(cnt_all, [iv, _splat_i32(0)])
        cc_col = plsc.load_gather(cnt_all, [iv, _splat_i32(1)])
        before = iv < s
        vstart = jnp.sum(jnp.where(before, vc_col, 0))
        cstart = jnp.sum(jnp.where(before, cc_col, 0))
        t_total = jnp.sum(vc_col)

        @pl.loop(0, GROUPS)
        def _adjust(g):
            sl = pl.ds(16 * g, 16)
            lpc[sl] = lpc[sl] + vstart
            lrank[sl] = lrank[sl] + (cstart - 1)

        pltpu.sync_copy(lpc.at[pl.ds(0, ROWS_PER_SUB)],
                        spm_pc.at[pl.ds(rowbase, ROWS_PER_SUB)])
        pltpu.sync_copy(lbm.at[pl.ds(0, ROWS_PER_SUB)],
                        spm_bm.at[pl.ds(rowbase, ROWS_PER_SUB)])
        pltpu.sync_copy(lrank.at[pl.ds(0, ROWS_PER_SUB)],
                        spm_rank.at[pl.ds(rowbase, ROWS_PER_SUB)])

        @pl.when(s == 15)
        def _tail():
            tail = ROWS_MAX - ROWS_PER_SUB
            src = pl.ds(ROWS_PER_SUB, tail)
            dst = pl.ds(16 * ROWS_PER_SUB, tail)
            pltpu.sync_copy(lpc.at[src], spm_pc.at[dst])
            pltpu.sync_copy(lbm.at[src], spm_bm.at[dst])
            pltpu.sync_copy(lrank.at[src], spm_rank.at[dst])

        plsc.subcore_barrier()

        pltpu.sync_copy(spm_pc, pc_all)
        pltpu.sync_copy(spm_bm, bm_all)
        pltpu.sync_copy(spm_rank, rank_all)

        def do_chunk(ch):
            jv = 16 * ch + iv
            r1, _c1 = _cfun(pc_all, bm_all, jv, t_total)
            rk = plsc.load_gather(rank_all, [r1])
            tm1 = jnp.maximum(t_total - 1, 0)
            keep = jnp.clip(rk, 0, tm1)
            r2, c2 = _cfun(pc_all, bm_all, keep, t_total)
            for t in range(5):
                plsc.store_scatter(idx3, [iv * 5 + t],
                                   5 * (b * N + r2) + t)
            c2buf[...] = c2
            pltpu.async_copy(pred_ref.at[idx3], rowbuf, sem).wait()

            @pl.loop(0, 16)
            def _det(d):
                base = 5 * d
                best = _splat_f32(NEG_INF)
                aidx = jnp.zeros((16,), jnp.int32)
                chunks = [(0, 20, 0, 0), (0, 24, 4, 12)]
                for r40 in range(1, 5):
                    kb = 40 * r40 - 20
                    chunks += [(r40, 0, kb, 0), (r40, 16, kb + 16, 0),
                               (r40, 24, kb + 24, 8)]
                for (r40, co, kb, lo) in chunks:
                    v = rowbuf[base + r40, pl.ds(co, 16)]
                    if lo:
                        v = jnp.where(iv >= lo, v, NEG_INF)
                    m = jnp.max(v)
                    eq = v == m
                    f = plsc.all_reduce_ffs(eq)
                    upd = m > best
                    aidx = jnp.where(upd, kb + f, aidx)
                    best = jnp.maximum(best, m)
                theta = (aidx.astype(jnp.float32) - 90.0) * jnp.float32(
                    0.017453292519943295)
                c2s = c2buf[d]
                coff = c2s.astype(jnp.float32) * MAX_WH
                v0 = rowbuf[base, pl.ds(0, 16)]
                v4 = rowbuf[base, pl.ds(4, 16)]
                conf_s = rowbuf[base, 4]
                score = jnp.sum(jnp.where(iv == 1 + c2s, v4, 0.0)) * conf_s
                outv = jnp.where(
                    iv < 4, v0 + coff,
                    jnp.where(iv == 4, theta,
                              jnp.where(iv == 5, score,
                                        jnp.where(iv == 6,
                                                  c2s.astype(jnp.float32),
                                                  0.0))))
                live = jnp.where(16 * ch + d < t_total,
                                 jnp.float32(1.0), jnp.float32(0.0))
                outbuf[d, :] = outv * live

            pltpu.sync_copy(outbuf,
                            out_ref.at[b].at[pl.ds(16 * ch, 16), :])

        do_chunk(s)

        @pl.when(s < NCHUNK - 16)
        def _extra():
            do_chunk(16 + s)

        plsc.subcore_barrier()


@jax.jit
def _nms_sc(pred40):
    mesh = plsc.VectorSubcoreMesh(core_axis_name="c", subcore_axis_name="s")
    kfn = pl.kernel(
        _sc_body,
        out_type=jax.ShapeDtypeStruct((B, DET_PAD, 16), jnp.float32),
        mesh=mesh,
        scratch_types=[
            pltpu.VMEM((ROWS_MAX, W), jnp.float32),
            pltpu.VMEM((ROWS_MAX,), jnp.int32),
            pltpu.VMEM((ROWS_MAX,), jnp.int32),
            pltpu.VMEM((ROWS_MAX,), jnp.int32),
            pltpu.VMEM((ROWS_MAX,), jnp.int32),
            pltpu.VMEM((N,), jnp.int32),
            pltpu.VMEM((N,), jnp.int32),
            pltpu.VMEM((N,), jnp.int32),
            pltpu.VMEM((80,), jnp.int32),
            pltpu.VMEM((80, W), jnp.float32),
            pltpu.VMEM((16, 16), jnp.float32),
            pltpu.VMEM((16,), jnp.int32),
            pltpu.VMEM((16,), jnp.int32),
            pltpu.VMEM((16, 16), jnp.int32),
            pltpu.VMEM_SHARED((N,), jnp.int32),
            pltpu.VMEM_SHARED((N,), jnp.int32),
            pltpu.VMEM_SHARED((N,), jnp.int32),
            pltpu.VMEM_SHARED((16, 16), jnp.int32),
            pltpu.SemaphoreType.DMA,
        ],
    )
    return kfn(pred40)


def kernel(x):
    pred40 = x.reshape(B * N * 5, W)
    outpad = _nms_sc(pred40)
    return outpad[:, :MAX_DET, :7]


# speedup vs baseline: 11.7806x; 11.7806x over previous
import dataclasses

import jax
import jax.numpy as jnp
from jax import lax
from jax.experimental import pallas as pl
from jax.experimental.pallas import tpu as pltpu
from jax.experimental.pallas import tpu_sc as plsc

CONF = 0.3
MAX_WH = 4096.0
MAX_DET = 300
N = 20000
NCLS = 15
B = 4
W = 40
ROWS_PER_SUB = 1248
ROWS_MAX = 1280
GROUPS = ROWS_MAX // 16
DET_PAD = 304
NCHUNK = DET_PAD // 16
NEG_INF = float("-inf")


def _iota():
    return lax.iota(jnp.int32, 16)


def _splat_i32(v):
    return jnp.zeros((16,), jnp.int32) + v


def _splat_f32(v):
    return jnp.zeros((16,), jnp.float32) + v


def _cfun(pc_ref, bm_ref, q, t_scalar):
    lo = jnp.zeros((16,), jnp.int32)
    hi = _splat_i32(N)
    nm1 = _splat_i32(N - 1)
    for _ in range(15):
        mid = jnp.minimum(lax.shift_right_logical(lo + hi, 1), nm1)
        pv = plsc.load_gather(pc_ref, [mid])
        cond = pv > q
        hi = jnp.where(cond, mid, hi)
        lo = jnp.where(cond, lo, mid + 1)
    in_t = q < t_scalar
    r = jnp.where(in_t, lo, 0)
    pcm1 = plsc.load_gather(pc_ref, [jnp.maximum(r - 1, 0)])
    pcx = jnp.where(r > 0, pcm1, 0)
    rem = q - pcx
    bmv = plsc.load_gather(bm_ref, [r])
    col = jnp.zeros((16,), jnp.int32)
    seen = jnp.zeros((16,), jnp.int32)
    found = jnp.zeros((16,), jnp.bool_)
    for t in range(NCLS):
        bit = lax.shift_right_logical(bmv, t) & 1
        hit = (bit == 1) & (seen == rem) & jnp.logical_not(found)
        col = jnp.where(hit, t, col)
        found = jnp.logical_or(found, hit)
        seen = seen + bit
    col = jnp.where(in_t, col, 0)
    return r, col


def _sc_body(pred_ref, out_ref, buf, idx1, lbm, lpc, lrank, pc_all, bm_all,
             rank_all, idx3, rowbuf, outbuf, c2buf, tot_v, cnt_all,
             spm_pc, spm_bm, spm_rank, spm_cnt, sem):
    c = lax.axis_index("c")
    s = lax.axis_index("s")
    iv = _iota()
    rowbase = ROWS_PER_SUB * s
    nrows = jnp.where(s == 15, ROWS_MAX, ROWS_PER_SUB)

    for bb in range(2):
        b = 2 * c + bb

        @pl.loop(0, GROUPS)
        def _build_idx(g):
            rid = iv + 16 * g
            idx1[pl.ds(16 * g, 16)] = 5 * (b * N + rowbase + rid)

        for t in range(ROWS_MAX // 128):
            pltpu.async_copy(
                pred_ref.at[idx1.at[pl.ds(128 * t, 128)]],
                buf.at[pl.ds(128 * t, 128), :], sem)
        for t in range(ROWS_MAX // 128):
            pltpu.make_async_copy(
                pred_ref.at[idx1.at[pl.ds(128 * t, 128)]],
                buf.at[pl.ds(128 * t, 128), :], sem).wait()

        def p1_group(g, carry):
            vcar, ccar = carry
            rid = iv + 16 * g
            rmask = rid < nrows
            confv = plsc.load_gather(buf, [rid, _splat_i32(4)])
            mc = (confv > CONF) & rmask
            bmv = jnp.zeros((16,), jnp.int32)
            cntv = jnp.zeros((16,), jnp.int32)
            for j in range(NCLS):
                cv = plsc.load_gather(buf, [rid, _splat_i32(5 + j)])
                vj = mc & (cv * confv > CONF)
                bmv = bmv | jnp.where(vj, 1 << j, 0)
                cntv = cntv + jnp.where(vj, 1, 0)
            lbm[pl.ds(16 * g, 16)] = bmv
            pcv = plsc.cumsum(cntv) + vcar
            lpc[pl.ds(16 * g, 16)] = pcv
            mci = jnp.where(mc, 1, 0)
            rkv = plsc.cumsum(mci) + ccar
            lrank[pl.ds(16 * g, 16)] = rkv
            return vcar + jnp.sum(cntv), ccar + jnp.sum(mci)

        vc_tot, cc_tot = lax.fori_loop(
            0, GROUPS, p1_group, (jnp.int32(0), jnp.int32(0)))

        totv = jnp.where(iv == 0, vc_tot, jnp.where(iv == 1, cc_tot, 0))
        tot_v[...] = totv
        pltpu.sync_copy(tot_v, spm_cnt.at[s])
        plsc.subcore_barrier()

        pltpu.sync_copy(spm_cnt, cnt_all)
        vc_col = plsc.load_gather(cnt_all, [iv, _splat_i32(0)])
        cc_col = plsc.load_gather(cnt_all, [iv, _splat_i32(1)])
        before = iv < s
        vstart = jnp.sum(jnp.where(before, vc_col, 0))
        cstart = jnp.sum(jnp.where(before, cc_col, 0))
        t_total = jnp.sum(vc_col)

        @pl.loop(0, GROUPS)
        def _adjust(g):
            sl = pl.ds(16 * g, 16)
            lpc[sl] = lpc[sl] + vstart
            lrank[sl] = lrank[sl] + (cstart - 1)

        pltpu.sync_copy(lpc.at[pl.ds(0, ROWS_PER_SUB)],
                        spm_pc.at[pl.ds(rowbase, ROWS_PER_SUB)])
        pltpu.sync_copy(lbm.at[pl.ds(0, ROWS_PER_SUB)],
                        spm_bm.at[pl.ds(rowbase, ROWS_PER_SUB)])
        pltpu.sync_copy(lrank.at[pl.ds(0, ROWS_PER_SUB)],
                        spm_rank.at[pl.ds(rowbase, ROWS_PER_SUB)])

        @pl.when(s == 15)
        def _tail():
            tail = ROWS_MAX - ROWS_PER_SUB
            src = pl.ds(ROWS_PER_SUB, tail)
            dst = pl.ds(16 * ROWS_PER_SUB, tail)
            pltpu.sync_copy(lpc.at[src], spm_pc.at[dst])
            pltpu.sync_copy(lbm.at[src], spm_bm.at[dst])
            pltpu.sync_copy(lrank.at[src], spm_rank.at[dst])

        plsc.subcore_barrier()

        pltpu.sync_copy(spm_pc, pc_all)
        pltpu.sync_copy(spm_bm, bm_all)
        pltpu.sync_copy(spm_rank, rank_all)

        def do_chunk(ch):
            jv = 16 * ch + iv
            r1, _c1 = _cfun(pc_all, bm_all, jv, t_total)
            rk = plsc.load_gather(rank_all, [r1])
            tm1 = jnp.maximum(t_total - 1, 0)
            keep = jnp.clip(rk, 0, tm1)
            r2, c2 = _cfun(pc_all, bm_all, keep, t_total)
            for t in range(5):
                plsc.store_scatter(idx3, [iv * 5 + t],
                                   5 * (b * N + r2) + t)
            c2buf[...] = c2
            pltpu.async_copy(pred_ref.at[idx3], rowbuf, sem).wait()

            @pl.loop(0, 16)
            def _det(d):
                base = 5 * d
                best = _splat_f32(NEG_INF)
                aidx = jnp.zeros((16,), jnp.int32)
                chunks = [(0, 20, 0, 0), (0, 24, 4, 12)]
                for r40 in range(1, 5):
                    kb = 40 * r40 - 20
                    chunks += [(r40, 0, kb, 0), (r40, 16, kb + 16, 0),
                               (r40, 24, kb + 24, 8)]
                for (r40, co, kb, lo) in chunks:
                    v = rowbuf[base + r40, pl.ds(co, 16)]
                    if lo:
                        v = jnp.where(iv >= lo, v, NEG_INF)
                    m = jnp.max(v)
                    eq = v == m
                    f = plsc.all_reduce_ffs(eq)
                    upd = m > best
                    aidx = jnp.where(upd, kb + f, aidx)
                    best = jnp.maximum(best, m)
                theta = (aidx.astype(jnp.float32) - 90.0) * jnp.float32(
                    0.017453292519943295)
                c2s = plsc.load_gather(c2buf, [_splat_i32(d)])
                coff = c2s.astype(jnp.float32) * MAX_WH
                v0 = rowbuf[base, pl.ds(0, 16)]
                v4 = rowbuf[base, pl.ds(4, 16)]
                conf_s = plsc.load_gather(rowbuf,
                                          [_splat_i32(base), _splat_i32(4)])
                score = jnp.sum(jnp.where(iv == 1 + c2s, v4, 0.0)) * conf_s
                outv = jnp.where(
                    iv < 4, v0 + coff,
                    jnp.where(iv == 4, theta,
                              jnp.where(iv == 5, score,
                                        jnp.where(iv == 6,
                                                  c2s.astype(jnp.float32),
                                                  0.0))))
                live = jnp.where(16 * ch + d < t_total,
                                 jnp.float32(1.0), jnp.float32(0.0))
                outbuf[d, :] = outv * live

            pltpu.sync_copy(outbuf,
                            out_ref.at[b].at[pl.ds(16 * ch, 16), :])

        do_chunk(s)

        @pl.when(s < NCHUNK - 16)
        def _extra():
            do_chunk(16 + s)

        plsc.subcore_barrier()


@jax.jit
def _nms_sc(pred40):
    mesh = plsc.VectorSubcoreMesh(core_axis_name="c", subcore_axis_name="s")
    cp = pltpu.CompilerParams()
    fields = pltpu.CompilerParams.__dataclass_fields__
    if "needs_layout_passes" in fields:
        cp = dataclasses.replace(cp, needs_layout_passes=False)
    if "use_tc_tiling_on_sc" in fields:
        cp = dataclasses.replace(cp, use_tc_tiling_on_sc=False)
    kfn = pl.kernel(
        _sc_body,
        out_type=jax.ShapeDtypeStruct((B, DET_PAD, 16), jnp.float32),
        mesh=mesh,
        scratch_types=[
            pltpu.VMEM((ROWS_MAX, W), jnp.float32),
            pltpu.VMEM((ROWS_MAX,), jnp.int32),
            pltpu.VMEM((ROWS_MAX,), jnp.int32),
            pltpu.VMEM((ROWS_MAX,), jnp.int32),
            pltpu.VMEM((ROWS_MAX,), jnp.int32),
            pltpu.VMEM((N,), jnp.int32),
            pltpu.VMEM((N,), jnp.int32),
            pltpu.VMEM((N,), jnp.int32),
            pltpu.VMEM((80,), jnp.int32),
            pltpu.VMEM((80, W), jnp.float32),
            pltpu.VMEM((16, 16), jnp.float32),
            pltpu.VMEM((16,), jnp.int32),
            pltpu.VMEM((16,), jnp.int32),
            pltpu.VMEM((16, 16), jnp.int32),
            pltpu.VMEM_SHARED((N,), jnp.int32),
            pltpu.VMEM_SHARED((N,), jnp.int32),
            pltpu.VMEM_SHARED((N,), jnp.int32),
            pltpu.VMEM_SHARED((16, 16), jnp.int32),
            pltpu.SemaphoreType.DMA,
        ],
        compiler_params=cp,
    )
    return kfn(pred40)


def kernel(x):
    pred40 = x.reshape(B * N * 5, W)
    outpad = _nms_sc(pred40)
    return outpad[:, :MAX_DET, :7]
